# baseline (speedup 1.0000x reference)
import functools

import jax
import jax.numpy as jnp
from jax import lax
from jax.experimental import pallas as pl
from jax.experimental.pallas import tpu as pltpu
from jax.experimental.pallas import tpu_sc as plsc

NUM_CORES = 2
NUM_SUBCORES = 16
NUM_WORKERS = NUM_CORES * NUM_SUBCORES
ROWS_PER_STEP = 16
D_PAD = 128


def _emb_call(n_seq, seq_len, d_pad):
    mesh = plsc.VectorSubcoreMesh(core_axis_name="c", subcore_axis_name="s")
    seq_per_worker = n_seq // NUM_WORKERS
    n_steps = seq_per_worker // ROWS_PER_STEP

    @functools.partial(
        pl.kernel,
        out_type=jax.ShapeDtypeStruct((n_seq, seq_len, d_pad), jnp.float32),
        mesh=mesh,
        scratch_types=[
            pltpu.VMEM((seq_per_worker, seq_len), jnp.int32),
            pltpu.VMEM((2, ROWS_PER_STEP, seq_len, d_pad), jnp.float32),
            pltpu.SemaphoreType.DMA,
            pltpu.SemaphoreType.DMA,
            pltpu.SemaphoreType.DMA,
            pltpu.SemaphoreType.DMA,
        ],
        compiler_params=pltpu.CompilerParams(use_tc_tiling_on_sc=False),
    )
    def emb(x_hbm, w_hbm, out_hbm, idx_v, rows_v, g0, g1, w0, w1):
        wid = lax.axis_index("s") * NUM_CORES + lax.axis_index("c")
        base = wid * seq_per_worker
        pltpu.sync_copy(x_hbm.at[pl.ds(base, seq_per_worker)], idx_v)
        gsems = (g0, g1)
        wsems = (w0, w1)

        def fire(s, buf):
            for m in range(ROWS_PER_STEP):
                pltpu.async_copy(
                    w_hbm.at[idx_v.at[s * ROWS_PER_STEP + m]],
                    rows_v.at[buf, m],
                    gsems[buf])

        def drain_gathers(buf):
            pltpu.make_async_copy(
                out_hbm.at[pl.ds(0, ROWS_PER_STEP)], rows_v.at[buf],
                gsems[buf]).wait()

        def wait_writeback(s, buf):
            pltpu.make_async_copy(
                rows_v.at[buf],
                out_hbm.at[pl.ds(base + s * ROWS_PER_STEP, ROWS_PER_STEP)],
                wsems[buf]).wait()

        def do_step(s, buf):
            drain_gathers(buf)
            pltpu.async_copy(
                rows_v.at[buf],
                out_hbm.at[pl.ds(base + s * ROWS_PER_STEP, ROWS_PER_STEP)],
                wsems[buf])
            nxt = buf ^ 1

            @pl.when(s > 0)
            def _():
                wait_writeback(s - 1, nxt)

            @pl.when(s + 1 < n_steps)
            def _():
                fire(s + 1, nxt)

        fire(0, 0)

        def body(i, carry):
            do_step(2 * i, 0)
            do_step(2 * i + 1, 1)
            return carry

        lax.fori_loop(0, n_steps // 2, body, 0)
        wait_writeback(n_steps - 1, (n_steps - 1) % 2)

    return emb


def kernel(x, weight):
    n_seq, seq_len = x.shape
    d = weight.shape[1]
    assert n_seq % (NUM_WORKERS * ROWS_PER_STEP) == 0
    wpad = jnp.pad(weight, ((0, 0), (0, D_PAD - d)))
    out = _emb_call(n_seq, seq_len, D_PAD)(x.astype(jnp.int32), wpad)
    return out[:, :, :d]


# speedup vs baseline: 1.2477x; 1.2477x over previous
import functools

import jax
import jax.numpy as jnp
from jax import lax
from jax.experimental import pallas as pl
from jax.experimental.pallas import tpu as pltpu
from jax.experimental.pallas import tpu_sc as plsc

NUM_CORES = 2
NUM_SUBCORES = 16
NUM_WORKERS = NUM_CORES * NUM_SUBCORES
CHUNK = 128
D_PAD = 128


def _emb_call(n_chunks):
    mesh = plsc.VectorSubcoreMesh(core_axis_name="c", subcore_axis_name="s")
    tok_per_worker = n_chunks * CHUNK

    @functools.partial(
        pl.kernel,
        out_type=jax.ShapeDtypeStruct((NUM_WORKERS * tok_per_worker, D_PAD),
                                      jnp.float32),
        mesh=mesh,
        scratch_types=[
            pltpu.VMEM((n_chunks, CHUNK), jnp.int32),
            pltpu.VMEM((2, CHUNK, D_PAD), jnp.float32),
            pltpu.SemaphoreType.DMA,
            pltpu.SemaphoreType.DMA,
            pltpu.SemaphoreType.DMA,
            pltpu.SemaphoreType.DMA,
        ],
    )
    def emb(idx_hbm, w_hbm, out_hbm, idx_v, rows_v, g0, g1, w0, w1):
        wid = lax.axis_index("s") * NUM_CORES + lax.axis_index("c")
        base = wid * tok_per_worker
        pltpu.sync_copy(idx_hbm.at[wid], idx_v)
        gsems = (g0, g1)
        wsems = (w0, w1)

        def fire(g, buf):
            pltpu.async_copy(
                w_hbm.at[idx_v.at[g]], rows_v.at[buf], gsems[buf])

        def drain_gather(buf):
            pltpu.make_async_copy(
                out_hbm.at[pl.ds(0, CHUNK)], rows_v.at[buf],
                gsems[buf]).wait()

        def wait_writeback(g, buf):
            pltpu.make_async_copy(
                rows_v.at[buf],
                out_hbm.at[pl.ds(base + g * CHUNK, CHUNK)],
                wsems[buf]).wait()

        def do_step(g, buf):
            drain_gather(buf)
            pltpu.async_copy(
                rows_v.at[buf],
                out_hbm.at[pl.ds(base + g * CHUNK, CHUNK)],
                wsems[buf])
            nxt = buf ^ 1

            @pl.when(g > 0)
            def _():
                wait_writeback(g - 1, nxt)

            @pl.when(g + 1 < n_chunks)
            def _():
                fire(g + 1, nxt)

        fire(0, 0)

        def body(i, carry):
            do_step(2 * i, 0)
            do_step(2 * i + 1, 1)
            return carry

        lax.fori_loop(0, n_chunks // 2, body, 0)
        wait_writeback(n_chunks - 1, (n_chunks - 1) % 2)

    return emb


ROW_BLK = 2048


def _transpose_pad(wt, n_rows, d):
    n_blk = (n_rows + ROW_BLK - 1) // ROW_BLK

    def body(in_ref, out_ref):
        out_ref[:, 0:d] = in_ref[...].T

    return pl.pallas_call(
        body,
        grid=(n_blk,),
        in_specs=[pl.BlockSpec((d, ROW_BLK), lambda i: (0, i))],
        out_specs=pl.BlockSpec((ROW_BLK, D_PAD), lambda i: (i, 0)),
        out_shape=jax.ShapeDtypeStruct((n_blk * ROW_BLK, D_PAD), jnp.float32),
    )(wt)


T1_BLK = 512


def _finalize(out2d, n_seq, seq_len, d):
    n_blk = n_seq // T1_BLK

    def body(in_ref, out_ref):
        xr = in_ref[...].reshape(T1_BLK, seq_len, D_PAD)
        for t2 in range(seq_len):
            out_ref[t2] = xr[:, t2, 0:d].T

    return pl.pallas_call(
        body,
        grid=(n_blk,),
        in_specs=[pl.BlockSpec((T1_BLK * seq_len, D_PAD), lambda i: (i, 0))],
        out_specs=pl.BlockSpec((seq_len, d, T1_BLK), lambda i: (0, 0, i)),
        out_shape=jax.ShapeDtypeStruct((seq_len, d, n_seq), jnp.float32),
    )(out2d)


def kernel(x, weight):
    n_seq, seq_len = x.shape
    n_rows, d = weight.shape
    n_tok = n_seq * seq_len
    assert n_tok % (NUM_WORKERS * CHUNK) == 0
    n_chunks = n_tok // (NUM_WORKERS * CHUNK)
    idx = x.reshape(NUM_WORKERS, n_chunks, CHUNK).astype(jnp.int32)
    wpad = _transpose_pad(weight.T, n_rows, d)
    out2d = _emb_call(n_chunks)(idx, wpad)
    o3 = _finalize(out2d, n_seq, seq_len, d)
    return jnp.transpose(o3, (2, 0, 1))
